# baseline (speedup 1.0000x reference)
import functools

import jax
import jax.numpy as jnp
from jax import lax
from jax.experimental import pallas as pl
from jax.experimental.pallas import tpu as pltpu
from jax.experimental.pallas import tpu_sc as plsc

N_NODES = 10000
N_EDGES = 320000
N_REL = 4

NC = 2
NS = 16
CH = 128
EPP = N_EDGES // N_REL // NS
B = 40
NBLK = EPP // B
NBUF = 5
RPT = 624
TAIL = N_NODES - NS * RPT
ZR = 24


def _sc_aggregate(n_chunks):
    mesh = plsc.VectorSubcoreMesh(core_axis_name="c", subcore_axis_name="s")

    @functools.partial(
        pl.kernel,
        out_type=[jax.ShapeDtypeStruct((NC, 2, N_NODES, CH), jnp.float32)
                  for _ in range(n_chunks)],
        mesh=mesh,
        scratch_types=(
            [pltpu.VMEM((EPP,), jnp.int32),
             pltpu.VMEM((NBLK, B), jnp.int32)]
            + [pltpu.VMEM((B, CH), jnp.float32) for _ in range(NBUF)]
            + [pltpu.VMEM((ZR, CH), jnp.float32),
               pltpu.VMEM_SHARED((N_NODES, CH), jnp.float32)]
            + [pltpu.SemaphoreType.DMA for _ in range(2 * NBUF + 1)]
        ),
    )
    def agg(*refs):
        tbls = refs[:n_chunks]
        src_hbm, dst_hbm = refs[n_chunks:n_chunks + 2]
        outs = refs[n_chunks + 2:2 * n_chunks + 2]
        scratch = refs[2 * n_chunks + 2:]
        stage_src, rows_idx = scratch[0], scratch[1]
        gbufs = scratch[2:2 + NBUF]
        zbuf, accum = scratch[2 + NBUF], scratch[3 + NBUF]
        gsems = scratch[4 + NBUF:4 + 2 * NBUF]
        ssems = scratch[4 + 2 * NBUF:4 + 3 * NBUF]
        psem = scratch[4 + 3 * NBUF]
        bufs = tuple(zip(gbufs, gsems, ssems))

        cid = lax.axis_index("c")
        sid = lax.axis_index("s")

        zv = jnp.zeros((16,), jnp.float32)
        cpv = CH // 16

        def zk(k, _):
            zbuf[k // cpv, pl.ds((k % cpv) * 16, 16)] = zv
            return 0
        lax.fori_loop(0, (ZR * CH) // 16, zk, 0)

        for rp in range(2):
            tix = (2 * cid + rp) * NS + sid
            pltpu.async_copy(src_hbm.at[pl.ds(tix * EPP, EPP)], stage_src, psem)
            pltpu.async_copy(dst_hbm.at[tix], rows_idx, psem)

            for c in range(n_chunks):
                for q in range(RPT // ZR):
                    pltpu.sync_copy(
                        zbuf, accum.at[pl.ds(sid * RPT + q * ZR, ZR)])

                @pl.when(sid == NS - 1)
                def _():
                    pltpu.sync_copy(zbuf.at[pl.ds(0, TAIL)],
                                    accum.at[pl.ds(NS * RPT, TAIL)])
                if c == 0:
                    pltpu.make_async_copy(
                        src_hbm.at[pl.ds(0, EPP)], stage_src, psem).wait()
                    pltpu.make_async_copy(dst_hbm.at[0], rows_idx, psem).wait()
                plsc.subcore_barrier()

                tbl = tbls[c]

                def fire_gather(b, gb, gs):
                    pltpu.async_copy(
                        tbl.at[stage_src.at[pl.ds(b * B, B)]], gb, gs)

                def wait_gather(gb, gs):
                    pltpu.make_async_copy(tbl.at[pl.ds(0, B)], gb, gs).wait()

                def handle_block(b, gb, gs, ss, refill):
                    wait_gather(gb, gs)
                    pltpu.async_copy(gb, accum.at[rows_idx.at[b]], ss, add=True)
                    pltpu.make_async_copy(gb, accum.at[pl.ds(0, B)], ss).wait()
                    if refill:
                        fire_gather(b + NBUF, gb, gs)

                for p, (gb, gs, ss) in enumerate(bufs):
                    fire_gather(p, gb, gs)

                def group(j, _):
                    for p, (gb, gs, ss) in enumerate(bufs):
                        handle_block(NBUF * j + p, gb, gs, ss, True)
                    return 0
                lax.fori_loop(0, NBLK // NBUF - 1, group, 0)
                for p, (gb, gs, ss) in enumerate(bufs):
                    handle_block(NBLK - NBUF + p, gb, gs, ss, False)
                plsc.subcore_barrier()

                pltpu.sync_copy(
                    accum.at[pl.ds(sid * RPT, RPT)],
                    outs[c].at[cid, rp, pl.ds(sid * RPT, RPT)],
                )

                @pl.when(sid == NS - 1)
                def _():
                    pltpu.sync_copy(accum.at[pl.ds(NS * RPT, TAIL)],
                                    outs[c].at[cid, rp, pl.ds(NS * RPT, TAIL)])
                plsc.subcore_barrier()

    return agg


def _tc_layer1(s1, h, wstack, b1):
    br = 1000
    grid = (N_NODES // br,)

    def body(s_ref, h_ref, w_ref, b_ref, oa_ref, ob_ref):
        acc = jnp.dot(h_ref[...], w_ref[N_REL], preferred_element_type=jnp.float32)
        for r in range(N_REL):
            acc += jnp.dot(s_ref[r], w_ref[r], preferred_element_type=jnp.float32)
        res = jnp.maximum(acc + b_ref[...], 0.0)
        oa_ref[...] = res[:, :CH]
        ob_ref[...] = res[:, CH:]

    return pl.pallas_call(
        body,
        grid=grid,
        in_specs=[
            pl.BlockSpec((N_REL, br, CH), lambda i: (0, i, 0)),
            pl.BlockSpec((br, CH), lambda i: (i, 0)),
            pl.BlockSpec((N_REL + 1, CH, 256), lambda i: (0, 0, 0)),
            pl.BlockSpec((1, 256), lambda i: (0, 0)),
        ],
        out_specs=[
            pl.BlockSpec((br, CH), lambda i: (i, 0)),
            pl.BlockSpec((br, CH), lambda i: (i, 0)),
        ],
        out_shape=[
            jax.ShapeDtypeStruct((N_NODES, CH), jnp.float32),
            jax.ShapeDtypeStruct((N_NODES, CH), jnp.float32),
        ],
    )(s1, h, wstack, b1)


def _tc_layer2(s2a, s2b, x1a, x1b, wa, wb, b2, c1wt, c1b, c2wt, c2b):
    br = 1000
    ngrid = N_NODES // br

    def body(sa_ref, sb_ref, xa_ref, xb_ref, wa_ref, wb_ref, b_ref,
             c1w_ref, c1b_ref, c2w_ref, c2b_ref, o1_ref, o2_ref, pool_ref):
        i = pl.program_id(0)
        acc = jnp.dot(xa_ref[...], wa_ref[N_REL], preferred_element_type=jnp.float32)
        acc += jnp.dot(xb_ref[...], wb_ref[N_REL], preferred_element_type=jnp.float32)
        for r in range(N_REL):
            acc += jnp.dot(sa_ref[r], wa_ref[r], preferred_element_type=jnp.float32)
            acc += jnp.dot(sb_ref[r], wb_ref[r], preferred_element_type=jnp.float32)
        x2 = jnp.maximum(acc + b_ref[...], 0.0)
        m = jnp.max(x2, axis=0, keepdims=True)

        @pl.when(i == 0)
        def _():
            pool_ref[...] = m

        @pl.when(i > 0)
        def _():
            pool_ref[...] = jnp.maximum(pool_ref[...], m)

        p = pool_ref[...]
        o1_ref[...] = jnp.dot(p, c1w_ref[...], preferred_element_type=jnp.float32) + c1b_ref[...]
        o2_ref[...] = jnp.dot(p, c2w_ref[...], preferred_element_type=jnp.float32) + c2b_ref[...]

    return pl.pallas_call(
        body,
        grid=(ngrid,),
        in_specs=[
            pl.BlockSpec((N_REL, br, CH), lambda i: (0, i, 0)),
            pl.BlockSpec((N_REL, br, CH), lambda i: (0, i, 0)),
            pl.BlockSpec((br, CH), lambda i: (i, 0)),
            pl.BlockSpec((br, CH), lambda i: (i, 0)),
            pl.BlockSpec((N_REL + 1, CH, 256), lambda i: (0, 0, 0)),
            pl.BlockSpec((N_REL + 1, CH, 256), lambda i: (0, 0, 0)),
            pl.BlockSpec((1, 256), lambda i: (0, 0)),
            pl.BlockSpec((256, 8), lambda i: (0, 0)),
            pl.BlockSpec((1, 8), lambda i: (0, 0)),
            pl.BlockSpec((256, 16), lambda i: (0, 0)),
            pl.BlockSpec((1, 16), lambda i: (0, 0)),
        ],
        out_specs=[
            pl.BlockSpec((1, 8), lambda i: (0, 0)),
            pl.BlockSpec((1, 16), lambda i: (0, 0)),
        ],
        out_shape=[
            jax.ShapeDtypeStruct((1, 8), jnp.float32),
            jax.ShapeDtypeStruct((1, 16), jnp.float32),
        ],
        scratch_shapes=[pltpu.VMEM((1, 256), jnp.float32)],
    )(s2a, s2b, x1a, x1b, wa, wb, b2, c1wt, c1b, c2wt, c2b)


@jax.jit
def kernel(h, edge_index, W1, loop_w1, b1, W2, loop_w2, b2,
           cls1_w, cls1_b, cls2_w, cls2_b):
    ei_p = edge_index.reshape(2, -1, N_REL).transpose(0, 2, 1)
    src_p = ei_p[0].reshape(-1)
    dst_p = ei_p[1].reshape(N_REL * NS, NBLK, B)

    s1_parts = _sc_aggregate(1)(h, src_p, dst_p)
    s1 = s1_parts[0].reshape(N_REL, N_NODES, CH)
    w1stack = jnp.concatenate([W1, loop_w1[None]], axis=0)
    x1a, x1b = _tc_layer1(s1, h, w1stack, b1.reshape(1, 256))

    s2a_p, s2b_p = _sc_aggregate(2)(x1a, x1b, src_p, dst_p)
    s2a = s2a_p.reshape(N_REL, N_NODES, CH)
    s2b = s2b_p.reshape(N_REL, N_NODES, CH)
    wa = jnp.concatenate([W2[:, :CH, :], loop_w2[None, :CH, :]], axis=0)
    wb = jnp.concatenate([W2[:, CH:, :], loop_w2[None, CH:, :]], axis=0)
    out1, out2 = _tc_layer2(
        s2a, s2b, x1a, x1b, wa, wb, b2.reshape(1, 256),
        cls1_w.T, cls1_b.reshape(1, 8), cls2_w.T, cls2_b.reshape(1, 16))
    return (out1, out2)


# speedup vs baseline: 1.0650x; 1.0650x over previous
import functools

import jax
import jax.numpy as jnp
from jax import lax
from jax.experimental import pallas as pl
from jax.experimental.pallas import tpu as pltpu
from jax.experimental.pallas import tpu_sc as plsc

N_NODES = 10000
N_EDGES = 320000
N_REL = 4

NC = 2
NS = 16
CH = 128
EPP = N_EDGES // N_REL // NS
B = 40
NBLK = EPP // B
NBUF = 5
RPT = 624
TAIL = N_NODES - NS * RPT
ZR = 16


def _sc_aggregate(n_chunks):
    mesh = plsc.VectorSubcoreMesh(core_axis_name="c", subcore_axis_name="s")

    @functools.partial(
        pl.kernel,
        out_type=[jax.ShapeDtypeStruct((NC, 2, N_NODES, CH), jnp.float32)
                  for _ in range(n_chunks)],
        mesh=mesh,
        scratch_types=(
            [pltpu.VMEM((EPP,), jnp.int32),
             pltpu.VMEM((NBLK, B), jnp.int32)]
            + [pltpu.VMEM((B, CH), jnp.float32) for _ in range(NBUF)]
            + [pltpu.VMEM((ZR, CH), jnp.float32),
               pltpu.VMEM_SHARED((N_NODES, CH), jnp.float32)]
            + [pltpu.SemaphoreType.DMA for _ in range(2 * NBUF)]
        ),
    )
    def agg(*refs):
        tbls = refs[:n_chunks]
        src_hbm, dst_hbm = refs[n_chunks:n_chunks + 2]
        outs = refs[n_chunks + 2:2 * n_chunks + 2]
        scratch = refs[2 * n_chunks + 2:]
        stage_src, rows_idx = scratch[0], scratch[1]
        gbufs = scratch[2:2 + NBUF]
        zbuf, accum = scratch[2 + NBUF], scratch[3 + NBUF]
        gsems = scratch[4 + NBUF:4 + 2 * NBUF]
        ssems = scratch[4 + 2 * NBUF:4 + 3 * NBUF]
        bufs = tuple(zip(gbufs, gsems, ssems))

        cid = lax.axis_index("c")
        sid = lax.axis_index("s")

        zv = jnp.zeros((16,), jnp.float32)
        cpv = CH // 16

        def zk(k, _):
            zbuf[k // cpv, pl.ds((k % cpv) * 16, 16)] = zv
            return 0
        lax.fori_loop(0, (ZR * CH) // 16, zk, 0)

        for rp in range(2):
            tix = (2 * cid + rp) * NS + sid
            pltpu.sync_copy(src_hbm.at[pl.ds(tix * EPP, EPP)], stage_src)
            pltpu.sync_copy(dst_hbm.at[tix], rows_idx)

            for c in range(n_chunks):
                for q in range(RPT // ZR):
                    pltpu.sync_copy(
                        zbuf, accum.at[pl.ds(sid * RPT + q * ZR, ZR)])

                @pl.when(sid == NS - 1)
                def _():
                    pltpu.sync_copy(zbuf.at[pl.ds(0, TAIL)],
                                    accum.at[pl.ds(NS * RPT, TAIL)])
                plsc.subcore_barrier()

                tbl = tbls[c]

                def fire_gather(b, gb, gs):
                    pltpu.async_copy(
                        tbl.at[stage_src.at[pl.ds(b * B, B)]], gb, gs)

                def wait_gather(gb, gs):
                    pltpu.make_async_copy(tbl.at[pl.ds(0, B)], gb, gs).wait()

                def handle_block(b, gb, gs, ss, refill):
                    wait_gather(gb, gs)
                    pltpu.async_copy(gb, accum.at[rows_idx.at[b]], ss, add=True)
                    pltpu.make_async_copy(gb, accum.at[pl.ds(0, B)], ss).wait()
                    if refill:
                        fire_gather(b + NBUF, gb, gs)

                for p, (gb, gs, ss) in enumerate(bufs):
                    fire_gather(p, gb, gs)

                def group(j, _):
                    for p, (gb, gs, ss) in enumerate(bufs):
                        handle_block(NBUF * j + p, gb, gs, ss, True)
                    return 0
                lax.fori_loop(0, NBLK // NBUF - 1, group, 0)
                for p, (gb, gs, ss) in enumerate(bufs):
                    handle_block(NBLK - NBUF + p, gb, gs, ss, False)
                plsc.subcore_barrier()

                pltpu.sync_copy(
                    accum.at[pl.ds(sid * RPT, RPT)],
                    outs[c].at[cid, rp, pl.ds(sid * RPT, RPT)],
                )

                @pl.when(sid == NS - 1)
                def _():
                    pltpu.sync_copy(accum.at[pl.ds(NS * RPT, TAIL)],
                                    outs[c].at[cid, rp, pl.ds(NS * RPT, TAIL)])
                plsc.subcore_barrier()

    return agg


def _tc_layer1(s1, h, wstack, b1):
    br = 1000
    grid = (N_NODES // br,)

    def body(s_ref, h_ref, w_ref, b_ref, oa_ref, ob_ref):
        acc = jnp.dot(h_ref[...], w_ref[N_REL], preferred_element_type=jnp.float32)
        for r in range(N_REL):
            acc += jnp.dot(s_ref[r], w_ref[r], preferred_element_type=jnp.float32)
        res = jnp.maximum(acc + b_ref[...], 0.0)
        oa_ref[...] = res[:, :CH]
        ob_ref[...] = res[:, CH:]

    return pl.pallas_call(
        body,
        grid=grid,
        in_specs=[
            pl.BlockSpec((N_REL, br, CH), lambda i: (0, i, 0)),
            pl.BlockSpec((br, CH), lambda i: (i, 0)),
            pl.BlockSpec((N_REL + 1, CH, 256), lambda i: (0, 0, 0)),
            pl.BlockSpec((1, 256), lambda i: (0, 0)),
        ],
        out_specs=[
            pl.BlockSpec((br, CH), lambda i: (i, 0)),
            pl.BlockSpec((br, CH), lambda i: (i, 0)),
        ],
        out_shape=[
            jax.ShapeDtypeStruct((N_NODES, CH), jnp.float32),
            jax.ShapeDtypeStruct((N_NODES, CH), jnp.float32),
        ],
    )(s1, h, wstack, b1)


def _tc_layer2(s2a, s2b, x1a, x1b, wa, wb, b2, c1wt, c1b, c2wt, c2b):
    br = 1000
    ngrid = N_NODES // br

    def body(sa_ref, sb_ref, xa_ref, xb_ref, wa_ref, wb_ref, b_ref,
             c1w_ref, c1b_ref, c2w_ref, c2b_ref, o1_ref, o2_ref, pool_ref):
        i = pl.program_id(0)
        acc = jnp.dot(xa_ref[...], wa_ref[N_REL], preferred_element_type=jnp.float32)
        acc += jnp.dot(xb_ref[...], wb_ref[N_REL], preferred_element_type=jnp.float32)
        for r in range(N_REL):
            acc += jnp.dot(sa_ref[r], wa_ref[r], preferred_element_type=jnp.float32)
            acc += jnp.dot(sb_ref[r], wb_ref[r], preferred_element_type=jnp.float32)
        x2 = jnp.maximum(acc + b_ref[...], 0.0)
        m = jnp.max(x2, axis=0, keepdims=True)

        @pl.when(i == 0)
        def _():
            pool_ref[...] = m

        @pl.when(i > 0)
        def _():
            pool_ref[...] = jnp.maximum(pool_ref[...], m)

        p = pool_ref[...]
        o1_ref[...] = jnp.dot(p, c1w_ref[...], preferred_element_type=jnp.float32) + c1b_ref[...]
        o2_ref[...] = jnp.dot(p, c2w_ref[...], preferred_element_type=jnp.float32) + c2b_ref[...]

    return pl.pallas_call(
        body,
        grid=(ngrid,),
        in_specs=[
            pl.BlockSpec((N_REL, br, CH), lambda i: (0, i, 0)),
            pl.BlockSpec((N_REL, br, CH), lambda i: (0, i, 0)),
            pl.BlockSpec((br, CH), lambda i: (i, 0)),
            pl.BlockSpec((br, CH), lambda i: (i, 0)),
            pl.BlockSpec((N_REL + 1, CH, 256), lambda i: (0, 0, 0)),
            pl.BlockSpec((N_REL + 1, CH, 256), lambda i: (0, 0, 0)),
            pl.BlockSpec((1, 256), lambda i: (0, 0)),
            pl.BlockSpec((256, 8), lambda i: (0, 0)),
            pl.BlockSpec((1, 8), lambda i: (0, 0)),
            pl.BlockSpec((256, 16), lambda i: (0, 0)),
            pl.BlockSpec((1, 16), lambda i: (0, 0)),
        ],
        out_specs=[
            pl.BlockSpec((1, 8), lambda i: (0, 0)),
            pl.BlockSpec((1, 16), lambda i: (0, 0)),
        ],
        out_shape=[
            jax.ShapeDtypeStruct((1, 8), jnp.float32),
            jax.ShapeDtypeStruct((1, 16), jnp.float32),
        ],
        scratch_shapes=[pltpu.VMEM((1, 256), jnp.float32)],
    )(s2a, s2b, x1a, x1b, wa, wb, b2, c1wt, c1b, c2wt, c2b)


@jax.jit
def kernel(h, edge_index, W1, loop_w1, b1, W2, loop_w2, b2,
           cls1_w, cls1_b, cls2_w, cls2_b):
    src_p = edge_index[0].reshape(-1, N_REL).T.reshape(-1)
    dst_p = edge_index[1].reshape(-1, N_REL).T.reshape(N_REL * NS, NBLK, B)

    s1_parts = _sc_aggregate(1)(h, src_p, dst_p)
    s1 = s1_parts[0].reshape(N_REL, N_NODES, CH)
    w1stack = jnp.concatenate([W1, loop_w1[None]], axis=0)
    x1a, x1b = _tc_layer1(s1, h, w1stack, b1.reshape(1, 256))

    s2a_p, s2b_p = _sc_aggregate(2)(x1a, x1b, src_p, dst_p)
    s2a = s2a_p.reshape(N_REL, N_NODES, CH)
    s2b = s2b_p.reshape(N_REL, N_NODES, CH)
    wa = jnp.concatenate([W2[:, :CH, :], loop_w2[None, :CH, :]], axis=0)
    wb = jnp.concatenate([W2[:, CH:, :], loop_w2[None, CH:, :]], axis=0)
    out1, out2 = _tc_layer2(
        s2a, s2b, x1a, x1b, wa, wb, b2.reshape(1, 256),
        cls1_w.T, cls1_b.reshape(1, 8), cls2_w.T, cls2_b.reshape(1, 16))
    return (out1, out2)
